# baseline (speedup 1.0000x reference)
import functools

import jax
import jax.numpy as jnp
from jax.experimental import pallas as pl
from jax.experimental.pallas import tpu as pltpu

_NUM_HEADS = 12
_CODEBOOK = 1024
_NUM_RES = 4
_HEAD_DIM = 64
_BGRP = 4


def _hvq_body(xt_ref, cb_ref, out_ref, idx_ref, perp_ref, cn_ref, cba_ref,
              counts_ref, *, n_tok, n_batch):
    bg = pl.program_id(0)
    hp = pl.program_id(1)
    m_rows = _BGRP * n_tok
    d = _HEAD_DIM

    @pl.when((hp == 0) & (bg == 0))
    def _init():
        counts_ref[...] = jnp.zeros_like(counts_ref)

    @pl.when(bg == 0)
    def _prep():
        for s in range(2):
            cbh = cb_ref[s]
            nrm = jnp.sqrt(jnp.sum(cbh * cbh, axis=1, keepdims=True))
            cn_ref[2 * hp + s] = (cbh / jnp.maximum(nrm, 1e-12)).astype(jnp.bfloat16)
            code = jax.lax.broadcasted_iota(jnp.int32, (_CODEBOOK, 1), 0)
            hi = (code // 16).astype(jnp.float32)
            lo = (code % 16).astype(jnp.float32)
            aug = jnp.concatenate(
                [cbh, hi, lo, jnp.zeros((_CODEBOOK, 62), jnp.float32)], axis=1)
            cba_ref[2 * hp + s] = aug.astype(jnp.bfloat16)

    ones_row = jnp.ones((1, m_rows), jnp.float32)
    pair = xt_ref[...].reshape(m_rows, 2 * d)
    accs = []
    for s in range(2):
        cn = cn_ref[2 * hp + s]
        cba = cba_ref[2 * hp + s]
        resid = pair[:, s * d:(s + 1) * d]
        acc = jnp.zeros_like(resid)
        for r in range(_NUM_RES):
            qn_nrm = jnp.sqrt(jnp.sum(resid * resid, axis=1, keepdims=True))
            qn = (resid / jnp.maximum(qn_nrm, 1e-12)).astype(jnp.bfloat16)
            sim = jax.lax.dot_general(
                qn, cn, (((1,), (1,)), ((), ())),
                preferred_element_type=jnp.float32)
            mx = jnp.max(sim, axis=1, keepdims=True)
            onehot = jnp.where(sim == mx, 1.0, 0.0).astype(jnp.bfloat16)
            qa = jnp.dot(onehot, cba, preferred_element_type=jnp.float32)
            quant = qa[:, :d]
            idx = (qa[:, d] * 16.0 + qa[:, d + 1]).astype(jnp.int32)
            acc = acc + quant
            resid = resid - quant
            idx_ref[:, s, :, r] = idx.reshape(_BGRP, n_tok)
            cnt = jax.lax.dot_general(
                ones_row, onehot, (((1,), (0,)), ((), ())),
                preferred_element_type=jnp.float32)
            counts_ref[2 * hp + s, r, :] = counts_ref[2 * hp + s, r, :] + cnt[0]
        accs.append(acc)
    acc2 = jnp.concatenate(accs, axis=1)
    out_ref[...] = acc2.reshape(_BGRP, n_tok, 2 * d)

    @pl.when((bg == pl.num_programs(0) - 1) & (hp == pl.num_programs(1) - 1))
    def _fin():
        mean = counts_ref[...] / float(n_batch * n_tok)
        ent = jnp.sum(mean * jnp.log(mean + 1e-10), axis=-1)
        perp_ref[...] = jnp.exp(-ent)


@jax.jit
def kernel(x, codebooks):
    bsz, n_tok, feat = x.shape
    h, m, d = codebooks.shape
    grid = (bsz // _BGRP, h // 2)
    out, idx_out, perp_out = pl.pallas_call(
        functools.partial(_hvq_body, n_tok=n_tok, n_batch=bsz),
        grid=grid,
        in_specs=[
            pl.BlockSpec((_BGRP, n_tok, 2 * d), lambda bb, hh: (bb, 0, hh)),
            pl.BlockSpec((2, m, d), lambda bb, hh: (hh, 0, 0)),
        ],
        out_specs=[
            pl.BlockSpec((_BGRP, n_tok, 2 * d), lambda bb, hh: (bb, 0, hh)),
            pl.BlockSpec((_BGRP, 2, n_tok, _NUM_RES), lambda bb, hh: (bb, hh, 0, 0)),
            pl.BlockSpec((h, _NUM_RES), lambda bb, hh: (0, 0)),
        ],
        out_shape=[
            jax.ShapeDtypeStruct((bsz, n_tok, feat), jnp.float32),
            jax.ShapeDtypeStruct((bsz, h, n_tok, _NUM_RES), jnp.int32),
            jax.ShapeDtypeStruct((h, _NUM_RES), jnp.float32),
        ],
        scratch_shapes=[
            pltpu.VMEM((h, m, d), jnp.bfloat16),
            pltpu.VMEM((h, m, 128), jnp.bfloat16),
            pltpu.VMEM((h, _NUM_RES, m), jnp.float32),
        ],
    )(x, codebooks)

    indices = idx_out.reshape(bsz, h, n_tok * _NUM_RES)
    perplexity = perp_out.reshape(h * _NUM_RES)
    return out, indices, perplexity


# speedup vs baseline: 1.8024x; 1.8024x over previous
import functools

import jax
import jax.numpy as jnp
from jax.experimental import pallas as pl
from jax.experimental.pallas import tpu as pltpu

_NUM_HEADS = 12
_CODEBOOK = 1024
_NUM_RES = 4
_HEAD_DIM = 64
_BGRP = 4


def _hvq_body(xt_ref, cb_ref, out_ref, idx_ref, perp_ref, cn_ref, cba_ref,
              counts_ref, *, n_tok, n_batch):
    bg = pl.program_id(0)
    hp = pl.program_id(1)
    m_rows = _BGRP * n_tok
    d = _HEAD_DIM

    @pl.when((hp == 0) & (bg == 0))
    def _init():
        counts_ref[...] = jnp.zeros_like(counts_ref)

    @pl.when(bg == 0)
    def _prep():
        for s in range(2):
            cbh = cb_ref[s]
            nrm = jnp.sqrt(jnp.sum(cbh * cbh, axis=1, keepdims=True))
            cn_ref[2 * hp + s] = (cbh / jnp.maximum(nrm, 1e-12)).astype(jnp.bfloat16)
            code = jax.lax.broadcasted_iota(jnp.int32, (_CODEBOOK, 1), 0)
            hi = (code // 16).astype(jnp.float32)
            lo = (code % 16).astype(jnp.float32)
            aug = jnp.concatenate(
                [cbh, hi, lo, jnp.zeros((_CODEBOOK, 62), jnp.float32)], axis=1)
            cba_ref[2 * hp + s] = aug.astype(jnp.bfloat16)

    ones_row = jnp.ones((1, m_rows), jnp.float32)
    accs = []
    for s in range(2):
        cn = cn_ref[2 * hp + s]
        cba = cba_ref[2 * hp + s]
        resid = xt_ref[:, s].reshape(m_rows, d)
        acc = jnp.zeros_like(resid)
        for r in range(_NUM_RES):
            qn_nrm = jnp.sqrt(jnp.sum(resid * resid, axis=1, keepdims=True))
            qn = (resid / jnp.maximum(qn_nrm, 1e-12)).astype(jnp.bfloat16)
            sim = jax.lax.dot_general(
                qn, cn, (((1,), (1,)), ((), ())),
                preferred_element_type=jnp.float32)
            mx = jnp.max(sim, axis=1, keepdims=True)
            onehot = jnp.where(sim == mx, 1.0, 0.0).astype(jnp.bfloat16)
            qa = jnp.dot(onehot, cba, preferred_element_type=jnp.float32)
            quant = qa[:, :d]
            idx = (qa[:, d] * 16.0 + qa[:, d + 1]).astype(jnp.int32)
            acc = acc + quant
            resid = resid - quant
            idx_ref[:, s, :, r] = idx.reshape(_BGRP, n_tok)
            cnt = jax.lax.dot_general(
                ones_row, onehot, (((1,), (0,)), ((), ())),
                preferred_element_type=jnp.float32)
            counts_ref[2 * hp + s, r, :] = counts_ref[2 * hp + s, r, :] + cnt[0]
        accs.append(acc)
    acc2 = jnp.concatenate(accs, axis=1)
    out_ref[:, :, pl.ds(hp * 2 * d, 2 * d)] = acc2.reshape(_BGRP, n_tok, 2 * d)

    @pl.when((bg == pl.num_programs(0) - 1) & (hp == pl.num_programs(1) - 1))
    def _fin():
        mean = counts_ref[...] / float(n_batch * n_tok)
        ent = jnp.sum(mean * jnp.log(mean + 1e-10), axis=-1)
        perp_ref[...] = jnp.exp(-ent)


@jax.jit
def kernel(x, codebooks):
    bsz, n_tok, feat = x.shape
    h, m, d = codebooks.shape
    xt = x.reshape(bsz, n_tok, h, d).transpose(0, 2, 1, 3)
    grid = (bsz // _BGRP, h // 2)
    out, idx_out, perp_out = pl.pallas_call(
        functools.partial(_hvq_body, n_tok=n_tok, n_batch=bsz),
        grid=grid,
        in_specs=[
            pl.BlockSpec((_BGRP, 2, n_tok, d), lambda bb, hh: (bb, hh, 0, 0)),
            pl.BlockSpec((2, m, d), lambda bb, hh: (hh, 0, 0)),
        ],
        out_specs=[
            pl.BlockSpec((_BGRP, n_tok, feat), lambda bb, hh: (bb, 0, 0)),
            pl.BlockSpec((_BGRP, 2, n_tok, _NUM_RES), lambda bb, hh: (bb, hh, 0, 0)),
            pl.BlockSpec((h, _NUM_RES), lambda bb, hh: (0, 0)),
        ],
        out_shape=[
            jax.ShapeDtypeStruct((bsz, n_tok, feat), jnp.float32),
            jax.ShapeDtypeStruct((bsz, h, n_tok, _NUM_RES), jnp.int32),
            jax.ShapeDtypeStruct((h, _NUM_RES), jnp.float32),
        ],
        scratch_shapes=[
            pltpu.VMEM((h, m, d), jnp.bfloat16),
            pltpu.VMEM((h, m, 128), jnp.bfloat16),
            pltpu.VMEM((h, _NUM_RES, m), jnp.float32),
        ],
    )(xt, codebooks)

    indices = idx_out.reshape(bsz, h, n_tok * _NUM_RES)
    perplexity = perp_out.reshape(h * _NUM_RES)
    return out, indices, perplexity
